# baseline (speedup 1.0000x reference)
import jax
import jax.numpy as jnp
from jax.experimental import pallas as pl
from jax.experimental.pallas import tpu as pltpu


def _body(winner_ref, mem_ref, val_ref, out_ref, sem):
    rows = out_ref.shape[0]

    def issue(g, carry):
        w = winner_ref[g]

        @pl.when(w >= 0)
        def _from_val():
            pltpu.make_async_copy(val_ref.at[w], out_ref.at[g], sem).start()

        @pl.when(w < 0)
        def _from_mem():
            pltpu.make_async_copy(mem_ref.at[g], out_ref.at[g], sem).start()

        return carry

    jax.lax.fori_loop(0, rows, issue, 0)

    def drain(g, carry):
        pltpu.make_async_copy(mem_ref.at[0], out_ref.at[0], sem).wait()
        return carry

    jax.lax.fori_loop(0, rows, drain, 0)


def kernel(mem, val, labels, slots):
    n_cls, length, n, c = mem.shape
    batch = val.shape[0]
    rows = n_cls * length

    ids = labels.astype(jnp.int32) * length + slots.astype(jnp.int32)
    matches = ids[None, :] == jnp.arange(rows, dtype=jnp.int32)[:, None]
    winner = jnp.max(
        jnp.where(matches, jnp.arange(batch, dtype=jnp.int32)[None, :], -1),
        axis=1,
    )

    out = pl.pallas_call(
        _body,
        grid_spec=pltpu.PrefetchScalarGridSpec(
            num_scalar_prefetch=1,
            grid=(1,),
            in_specs=[
                pl.BlockSpec(memory_space=pl.ANY),
                pl.BlockSpec(memory_space=pl.ANY),
            ],
            out_specs=pl.BlockSpec(memory_space=pl.ANY),
            scratch_shapes=[pltpu.SemaphoreType.DMA],
        ),
        out_shape=jax.ShapeDtypeStruct((rows, n, c), mem.dtype),
    )(winner, mem.reshape(rows, n, c), val)
    return out.reshape(mem.shape)


# speedup vs baseline: 43.4677x; 43.4677x over previous
import jax
import jax.numpy as jnp
from jax.experimental import pallas as pl
from jax.experimental.pallas import tpu as pltpu


_ROWS_PER_BLOCK = 100


def _body(winner_ref, mem_ref, val_ref, out_ref):
    g = pl.program_id(0)
    out_ref[...] = mem_ref[...]
    for r in range(_ROWS_PER_BLOCK):
        w = winner_ref[g * _ROWS_PER_BLOCK + r]

        @pl.when(w >= 0)
        def _use_val(w=w, r=r):
            out_ref[r] = val_ref[w]


def kernel(mem, val, labels, slots):
    n_cls, length, n, c = mem.shape
    batch = val.shape[0]
    rows = n_cls * length

    ids = labels.astype(jnp.int32) * length + slots.astype(jnp.int32)
    matches = ids[None, :] == jnp.arange(rows, dtype=jnp.int32)[:, None]
    winner = jnp.max(
        jnp.where(matches, jnp.arange(batch, dtype=jnp.int32)[None, :], -1),
        axis=1,
    )

    out = pl.pallas_call(
        _body,
        grid_spec=pltpu.PrefetchScalarGridSpec(
            num_scalar_prefetch=1,
            grid=(rows // _ROWS_PER_BLOCK,),
            in_specs=[
                pl.BlockSpec((_ROWS_PER_BLOCK, n, c), lambda g, w_ref: (g, 0, 0)),
                pl.BlockSpec((batch, n, c), lambda g, w_ref: (0, 0, 0)),
            ],
            out_specs=pl.BlockSpec((_ROWS_PER_BLOCK, n, c), lambda g, w_ref: (g, 0, 0)),
        ),
        out_shape=jax.ShapeDtypeStruct((rows, n, c), mem.dtype),
    )(winner, mem.reshape(rows, n, c), val)
    return out.reshape(mem.shape)
